# baseline (speedup 1.0000x reference)
import functools

import jax
import jax.numpy as jnp
from jax import lax
from jax.experimental import pallas as pl
from jax.experimental.pallas import tpu as pltpu
from jax.experimental.pallas import tpu_sc as plsc

_K1 = 21.7
_K2 = 40.0
_LN2 = 0.6931471805599453
_H = 512
_W = 512
_N = 128
_NW = 32
_RPW = _H // _NW
_SENT_LO = -1.0e4
_SENT_HI = 1.0e5
_EMPTY_THR = 1.0e6
_ACC_INIT = 3.0e10


@functools.cache
def _sc_partials_fn():
    mesh = plsc.VectorSubcoreMesh(core_axis_name="c", subcore_axis_name="s")
    return pl.kernel(
        _sc_partials,
        mesh=mesh,
        compiler_params=pltpu.CompilerParams(needs_layout_passes=False),
        out_type=jax.ShapeDtypeStruct((_NW, 5, _N), jnp.float32),
        scratch_types=[
            pltpu.VMEM((_RPW, _W), jnp.float32),
            pltpu.VMEM((_RPW, _W), jnp.float32),
            pltpu.VMEM((_RPW, _W), jnp.float32),
            pltpu.VMEM((_N, 2), jnp.int32),
            pltpu.VMEM((5, _N), jnp.float32),
        ],
    )


def _sc_partials(hd_hbm, pred_hbm, out_hbm, blk_v, sout_v, sin_v, pred_v,
                 parts_v):
    w = lax.axis_index("s") * 2 + lax.axis_index("c")
    r0 = _RPW * w
    pltpu.sync_copy(hd_hbm.at[pl.ds(pl.multiple_of(r0, 8), _RPW)], blk_v)
    pltpu.sync_copy(pred_hbm, pred_v)

    iota16 = lax.iota(jnp.int32, 16)
    zeros16 = jnp.zeros((16,), jnp.int32)

    def fwd_body(c, carry):
        fin, fout = carry
        m = plsc.load_gather(blk_v, [iota16, jnp.full((16,), c, jnp.int32)])
        cf = jnp.full((16,), c.astype(jnp.float32), jnp.float32)
        fin = jnp.where(m != 0.0, cf, fin)
        fout = jnp.where(m == 0.0, cf, fout)
        plsc.store_scatter(sin_v, [iota16, jnp.full((16,), c, jnp.int32)], fin)
        plsc.store_scatter(sout_v, [iota16, jnp.full((16,), c, jnp.int32)], fout)
        return fin, fout

    lo = jnp.full((16,), _SENT_LO, jnp.float32)
    lax.fori_loop(0, _W, fwd_body, (lo, lo))

    def bwd_body(t, carry):
        bin_, bout = carry
        c = _W - 1 - t
        ci = jnp.full((16,), c, jnp.int32)
        m = plsc.load_gather(blk_v, [iota16, ci])
        cf = jnp.full((16,), c.astype(jnp.float32), jnp.float32)
        bin_ = jnp.where(m != 0.0, cf, bin_)
        bout = jnp.where(m == 0.0, cf, bout)
        fin = plsc.load_gather(sin_v, [iota16, ci])
        fout = plsc.load_gather(sout_v, [iota16, ci])
        dfi = cf - fin
        dbi = bin_ - cf
        dfo = cf - fout
        dbo = bout - cf
        sin = jnp.minimum(dfi * dfi, dbi * dbi)
        sout = jnp.minimum(dfo * dfo, dbo * dbo)
        plsc.store_scatter(sin_v, [iota16, ci], sin)
        plsc.store_scatter(sout_v, [iota16, ci], sout)
        return bin_, bout

    hi = jnp.full((16,), _SENT_HI, jnp.float32)
    lax.fori_loop(0, _W, bwd_body, (hi, hi))

    for g in range(_N // 16):
        pidx = iota16 + (g * 16)
        p0v = plsc.load_gather(pred_v, [pidx, zeros16])
        p1v = plsc.load_gather(pred_v, [pidx, zeros16 + 1])
        p0f = p0v.astype(jnp.float32)
        p1f = p1v.astype(jnp.float32)
        acc_o = jnp.full((16,), _ACC_INIT, jnp.float32)
        acc_i = jnp.full((16,), _ACC_INIT, jnp.float32)
        for r in range(_RPW):
            rf = jnp.full((16,), (r0 + r).astype(jnp.float32), jnp.float32)
            dr = rf - p0f
            q = dr * dr
            so = plsc.load_gather(sout_v, [jnp.full((16,), r, jnp.int32), p1v])
            si = plsc.load_gather(sin_v, [jnp.full((16,), r, jnp.int32), p1v])
            acc_o = jnp.minimum(acc_o, q + so)
            acc_i = jnp.minimum(acc_i, q + si)
        rel = p0v - r0
        nbacc = jnp.zeros((16,), jnp.float32)
        for dr in (-1, 0):
            rr = rel + dr
            okr = (rr >= 0) & (rr < _RPW)
            rrc = jnp.clip(rr, 0, _RPW - 1)
            for dc in (-1, 0):
                cc = p1v + dc
                okc = cc >= 0
                ccc = jnp.maximum(cc, 0)
                v = plsc.load_gather(blk_v, [rrc, ccc])
                nbacc = jnp.maximum(nbacc, jnp.where(okr & okc, v, 0.0))
        parts_v[0, pl.ds(g * 16, 16)] = acc_o
        parts_v[1, pl.ds(g * 16, 16)] = acc_i
        parts_v[2, pl.ds(g * 16, 16)] = -nbacc
        parts_v[3, pl.ds(g * 16, 16)] = p0f
        parts_v[4, pl.ds(g * 16, 16)] = p1f
    pltpu.sync_copy(parts_v, out_hbm.at[w])


def _merge_body(parts_ref, out_ref):
    parts = parts_ref[...]
    red = jnp.min(parts, axis=0, keepdims=True)
    m2o = red[:, 0:_N]
    m2i = red[:, _N:2 * _N]
    nbm = -red[:, 2 * _N:3 * _N]
    p0f = red[:, 3 * _N:4 * _N]
    p1f = red[:, 4 * _N:5 * _N]

    fb = p0f * p0f + p1f * p1f
    m2o = jnp.where(m2o > _EMPTY_THR, fb, m2o)
    m2i = jnp.where(m2i > _EMPTY_THR, fb, m2i)

    anyn = (nbm > 0.5) & (p0f >= 1.0) & (p1f >= 1.0)
    valid = ((p0f >= 0.0) & (p0f <= float(_H))
             & (p1f >= 0.0) & (p1f <= float(_W)))
    loss = jnp.where(anyn, jnp.exp(jnp.sqrt(m2o) * (_LN2 / _K2)) - 1.0,
                     jnp.exp(m2i * (-1.0 / _K1)))
    loss = jnp.where(valid, loss, 0.0)
    out_ref[...] = jnp.sum(loss, keepdims=True).reshape(1, 1) * (1.0 / _N)


def kernel(hd_map, prediction):
    parts = _sc_partials_fn()(hd_map, prediction.astype(jnp.int32))
    out = pl.pallas_call(
        _merge_body,
        out_shape=jax.ShapeDtypeStruct((1, 1), jnp.float32),
    )(parts.reshape(_NW, 5 * _N))
    return out[0, 0]


# speedup vs baseline: 2.3107x; 2.3107x over previous
import functools

import jax
import jax.numpy as jnp
from jax import lax
from jax.experimental import pallas as pl
from jax.experimental.pallas import tpu as pltpu
from jax.experimental.pallas import tpu_sc as plsc

_K1 = 21.7
_K2 = 40.0
_LN2 = 0.6931471805599453
_H = 512
_W = 512
_N = 128
_NW = 16
_PPW = _N // _NW
_SENT_LO = -1.0e4
_SENT_HI = 1.0e5
_EMPTY_THR = 1.0e6
_ACC_INIT = 3.0e10


def _tables_body(hd_ref, out_ref):
    hd = hd_ref[...]
    rowf = lax.broadcasted_iota(jnp.int32, (_H, _W), 0).astype(jnp.float32)

    def table(mask):
        fwd = jnp.where(mask, rowf, _SENT_LO)
        bwd = jnp.where(mask, rowf, _SENT_HI)
        k = 1
        for _ in range(9):
            top = jnp.full((k, _W), _SENT_LO, jnp.float32)
            fwd = jnp.maximum(fwd, jnp.concatenate([top, fwd[:_H - k, :]], axis=0))
            bot = jnp.full((k, _W), _SENT_HI, jnp.float32)
            bwd = jnp.minimum(bwd, jnp.concatenate([bwd[k:, :], bot], axis=0))
            k *= 2
        return jnp.minimum((rowf - fwd) ** 2, (bwd - rowf) ** 2)

    t_in = table(hd != 0.0)
    t_out = table(hd == 0.0)

    shifted = jnp.concatenate([jnp.zeros((1, _W), jnp.float32), hd[:_H - 1, :]], axis=0)
    rmax = jnp.maximum(hd, shifted)
    shiftc = jnp.concatenate([jnp.zeros((_H, 1), jnp.float32), rmax[:, :_W - 1]], axis=1)
    nb = jnp.maximum(rmax, shiftc)

    out_ref[...] = jnp.concatenate([t_out, t_in, nb], axis=1)


def _sqrt16(x):
    bits = plsc.bitcast(x, jnp.int32)
    seed = lax.shift_right_logical(bits, 1) + jnp.int32(0x1FBD1DF5)
    y = plsc.bitcast(seed, jnp.float32)
    for _ in range(3):
        y = 0.5 * (y + x / y)
    return y


@functools.cache
def _sc_points_fn():
    mesh = plsc.VectorSubcoreMesh(
        core_axis_name="c", subcore_axis_name="s", num_cores=1)
    return pl.kernel(
        _sc_points,
        mesh=mesh,
        compiler_params=pltpu.CompilerParams(needs_layout_passes=False),
        out_type=jax.ShapeDtypeStruct((_NW, 16), jnp.float32),
        scratch_types=[
            pltpu.VMEM((_PPW, 2), jnp.int32),
            pltpu.VMEM((16,), jnp.int32),
            pltpu.VMEM((_PPW, 3 * _W), jnp.float32),
            pltpu.VMEM((16,), jnp.float32),
            pltpu.VMEM((16, 16), jnp.float32),
            pltpu.VMEM_SHARED((16, 16), jnp.float32),
            pltpu.SemaphoreType.DMA,
        ],
    )


def _sc_points(table_hbm, pred_hbm, out_hbm, predw_v, idx_v, rows_v, out_v,
               acc_v, shared_v, sem):
    w = lax.axis_index("s")
    pltpu.sync_copy(pred_hbm.at[pl.ds(pl.multiple_of(_PPW * w, 8), _PPW)],
                    predw_v)

    iota16 = lax.iota(jnp.int32, 16)
    zeros16 = jnp.zeros((16,), jnp.int32)
    sel = jnp.minimum(iota16, _PPW - 1)
    idx_v[...] = plsc.load_gather(predw_v, [sel, zeros16])
    pltpu.async_copy(table_hbm.at[idx_v.at[pl.ds(0, _PPW)]], rows_v, sem).wait()

    wsum = jnp.zeros((16,), jnp.float32)
    for i in range(_PPW):
        isplat = jnp.full((16,), i, jnp.int32)
        p0v = plsc.load_gather(predw_v, [isplat, zeros16])
        p1v = plsc.load_gather(predw_v, [isplat, zeros16 + 1])
        p0f = p0v.astype(jnp.float32)
        p1f = p1v.astype(jnp.float32)
        acc_o = jnp.full((16,), _ACC_INIT, jnp.float32)
        acc_i = jnp.full((16,), _ACC_INIT, jnp.float32)
        for j in range(_W // 16):
            col = iota16 + (j * 16)
            d = col.astype(jnp.float32) - p1f
            qv = d * d
            t_o = rows_v[i, pl.ds(j * 16, 16)]
            t_i = rows_v[i, pl.ds(_W + j * 16, 16)]
            acc_o = jnp.minimum(acc_o, qv + t_o)
            acc_i = jnp.minimum(acc_i, qv + t_i)
        nbv = plsc.load_gather(rows_v, [isplat, p1v + 2 * _W])
        m2o = jnp.full((16,), jnp.min(acc_o), jnp.float32)
        m2i = jnp.full((16,), jnp.min(acc_i), jnp.float32)
        fb = p0f * p0f + p1f * p1f
        m2o = jnp.where(m2o > _EMPTY_THR, fb, m2o)
        m2i = jnp.where(m2i > _EMPTY_THR, fb, m2i)
        anyv = (nbv > 0.5) & (p0v >= 1) & (p1v >= 1)
        validv = (p0v >= 0) & (p0v <= _H) & (p1v >= 0) & (p1v <= _W)
        loss = jnp.where(anyv,
                         jnp.exp(_sqrt16(m2o) * (_LN2 / _K2)) - 1.0,
                         jnp.exp(m2i * (-1.0 / _K1)))
        wsum = wsum + jnp.where(validv, loss, 0.0)

    out_v[...] = wsum
    pltpu.sync_copy(out_v, out_hbm.at[w])


def kernel(hd_map, prediction):
    table = pl.pallas_call(
        _tables_body,
        out_shape=jax.ShapeDtypeStruct((_H, 3 * _W), jnp.float32),
    )(hd_map)
    sums = _sc_points_fn()(table, prediction.astype(jnp.int32))
    return jnp.sum(sums[:, 0]) * (1.0 / _N)
